# Initial kernel scaffold; baseline (speedup 1.0000x reference)
import jax
import jax.numpy as jnp
from jax.experimental import pallas as pl


def kernel(x, table):
    raise NotImplementedError("write your pallas kernel here")


# speedup vs baseline: 1.4818x; 1.4818x over previous
import functools

import jax
import jax.numpy as jnp
from jax import lax
from jax.experimental import pallas as pl
from jax.experimental.pallas import tpu as pltpu
from jax.experimental.pallas import tpu_sc as plsc

_VOCAB = 1000000
_D = 32
_B = 4096
_H = 200
_N = _B * _H

_NC = 2
_NS = 16
_NW = _NC * _NS
_PER_W = _N // _NW
_IDX_ROW = 128
_ROWS = _PER_W // _IDX_ROW
_G = 10
_CHUNK = _G * _IDX_ROW
_NG = _ROWS // _G


def _emb_body(idx_hbm, table_hbm, out_hbm, idx_v, rows_v, gsem, wsem):
    c = lax.axis_index("c")
    s = lax.axis_index("s")
    wid = s * _NC + c

    pltpu.sync_copy(idx_hbm.at[wid], idx_v)

    def group(g, _):
        handles = []
        for u in range(_G):
            h = pltpu.async_copy(
                table_hbm.at[idx_v.at[g * _G + u]],
                rows_v.at[pl.ds(u * _IDX_ROW, _IDX_ROW)],
                gsem,
            )
            handles.append(h)
        for h in handles:
            h.wait()
        pltpu.sync_copy(rows_v, out_hbm.at[wid, pl.ds(g * _CHUNK, _CHUNK)])
        return ()

    lax.fori_loop(0, _NG, group, (), unroll=False)


@jax.jit
def _emb(idx, table):
    mesh = plsc.VectorSubcoreMesh(
        core_axis_name="c", subcore_axis_name="s",
        num_cores=_NC, num_subcores=_NS,
    )
    f = pl.kernel(
        _emb_body,
        out_type=jax.ShapeDtypeStruct((_NW, _PER_W, _D), jnp.float32),
        mesh=mesh,
        scratch_types=[
            pltpu.VMEM((_ROWS, _IDX_ROW), jnp.int32),
            pltpu.VMEM((_CHUNK, _D), jnp.float32),
            pltpu.SemaphoreType.DMA,
            pltpu.SemaphoreType.DMA,
        ],
        compiler_params=pltpu.CompilerParams(use_tc_tiling_on_sc=False),
    )
    return f(idx, table)


def kernel(x, table):
    idx = x.astype(jnp.int32).reshape(_NW, _ROWS, _IDX_ROW)
    out = _emb(idx, table)
    return out.reshape(_B, _H, _D)
